# Initial kernel scaffold; baseline (speedup 1.0000x reference)
import jax
import jax.numpy as jnp
from jax.experimental import pallas as pl


def kernel(x, table):
    raise NotImplementedError("write your pallas kernel here")


# speedup vs baseline: 22.9955x; 22.9955x over previous
import jax
import jax.numpy as jnp
from jax.experimental import pallas as pl
from jax.experimental.pallas import tpu as pltpu

_BATCH = 16384
_HIST = 200
_FEAT = 128
_BLOCK = 64


def _body(x_ref, tab_ref, o_ref):
    xf = x_ref[...].astype(jnp.float32)
    t0 = tab_ref[0, :]
    dt = tab_ref[1, :] - t0
    xf3 = jax.lax.broadcast_in_dim(xf, (_BLOCK, _HIST, _FEAT), (0, 1))
    o_ref[...] = t0[None, None, :] + xf3 * dt[None, None, :]


def kernel(x, table):
    grid = (_BATCH // _BLOCK,)
    return pl.pallas_call(
        _body,
        grid=grid,
        in_specs=[
            pl.BlockSpec((_BLOCK, _HIST), lambda i: (i, 0)),
            pl.BlockSpec((2, _FEAT), lambda i: (0, 0)),
        ],
        out_specs=pl.BlockSpec((_BLOCK, _HIST, _FEAT), lambda i: (i, 0, 0)),
        out_shape=jax.ShapeDtypeStruct((_BATCH, _HIST, _FEAT), jnp.float32),
        compiler_params=pltpu.CompilerParams(
            dimension_semantics=("arbitrary",),
        ),
    )(x, table)
